# baseline (speedup 1.0000x reference)
import jax
import jax.numpy as jnp
from jax.experimental import pallas as pl
from jax.experimental.pallas import tpu as pltpu


def _fused_moe_kernel(x_ref, gwt_ref, gb_ref, wt_ref, b_ref, o_ref):
    xb = x_ref[...].astype(jnp.bfloat16)
    logits = jnp.dot(xb, gwt_ref[...], preferred_element_type=jnp.float32)
    logits = logits + gb_ref[...]
    m = jnp.max(logits, axis=1, keepdims=True)
    scale = 1.0 / jnp.sum(jnp.exp(logits - m), axis=1, keepdims=True)
    out = jnp.dot(xb, wt_ref[...], preferred_element_type=jnp.float32)
    o_ref[...] = (out + b_ref[...]) * scale


def _run(hs, gate_wt, gate_b2, expert_wt, expert_b2, *, tile_n, interpret=False):
    n, d = hs.shape
    e = gate_wt.shape[1]
    grid = (n // tile_n,)
    return pl.pallas_call(
        _fused_moe_kernel,
        grid=grid,
        in_specs=[
            pl.BlockSpec((tile_n, d), lambda i: (i, 0)),
            pl.BlockSpec((d, e), lambda i: (0, 0)),
            pl.BlockSpec((1, e), lambda i: (0, 0)),
            pl.BlockSpec((d, d), lambda i: (0, 0)),
            pl.BlockSpec((1, d), lambda i: (0, 0)),
        ],
        out_specs=pl.BlockSpec((tile_n, d), lambda i: (i, 0)),
        out_shape=jax.ShapeDtypeStruct((n, d), jnp.float32),
        interpret=interpret,
    )(hs, gate_wt, gate_b2, expert_wt, expert_b2)


def kernel(x, gate_w, gate_b, expert_w, expert_b):
    b, s, d = x.shape
    hs = x.reshape(b * s, d)
    out = _run(
        hs,
        gate_w.T.astype(jnp.bfloat16),
        gate_b.reshape(1, -1),
        expert_w.T.astype(jnp.bfloat16),
        expert_b.reshape(1, -1),
        tile_n=1024,
    )
    return out.reshape(b, s, d)


# speedup vs baseline: 1.1602x; 1.1602x over previous
import jax
import jax.numpy as jnp
from jax.experimental import pallas as pl
from jax.experimental.pallas import tpu as pltpu


def _fused_moe_kernel(x_ref, gw_ref, gb_ref, w_ref, b_ref, o_ref):
    xb = x_ref[...].astype(jnp.bfloat16)
    logits = jax.lax.dot_general(
        xb, gw_ref[...].astype(jnp.bfloat16), (((1,), (1,)), ((), ())),
        preferred_element_type=jnp.float32,
    ) + gb_ref[...]
    m = jnp.max(logits, axis=1, keepdims=True)
    scale = 1.0 / jnp.sum(jnp.exp(logits - m), axis=1, keepdims=True)
    out = jax.lax.dot_general(
        xb, w_ref[...].astype(jnp.bfloat16), (((1,), (1,)), ((), ())),
        preferred_element_type=jnp.float32,
    )
    o_ref[...] = (out + b_ref[...]) * scale


def _run(hs, gate_w, gate_b2, expert_w, expert_b2, *, tile_n, interpret=False):
    n, d = hs.shape
    e = gate_w.shape[0]
    grid = (n // tile_n,)
    return pl.pallas_call(
        _fused_moe_kernel,
        grid=grid,
        in_specs=[
            pl.BlockSpec((tile_n, d), lambda i: (i, 0)),
            pl.BlockSpec((e, d), lambda i: (0, 0)),
            pl.BlockSpec((1, e), lambda i: (0, 0)),
            pl.BlockSpec((d, d), lambda i: (0, 0)),
            pl.BlockSpec((1, d), lambda i: (0, 0)),
        ],
        out_specs=pl.BlockSpec((tile_n, d), lambda i: (i, 0)),
        out_shape=jax.ShapeDtypeStruct((n, d), jnp.float32),
        interpret=interpret,
    )(hs, gate_w, gate_b2, expert_w, expert_b2)


def kernel(x, gate_w, gate_b, expert_w, expert_b):
    b, s, d = x.shape
    hs = x.reshape(b * s, d)
    out = _run(
        hs,
        gate_w,
        gate_b.reshape(1, -1),
        expert_w,
        expert_b.reshape(1, -1),
        tile_n=512,
    )
    return out.reshape(b, s, d)


# speedup vs baseline: 1.1640x; 1.0032x over previous
import jax
import jax.numpy as jnp
from jax.experimental import pallas as pl
from jax.experimental.pallas import tpu as pltpu


def _fused_moe_kernel(x_ref, gw_ref, gb_ref, w_ref, b_ref, o_ref):
    xb = x_ref[...].astype(jnp.bfloat16)
    logits = jax.lax.dot_general(
        xb, gw_ref[...].astype(jnp.bfloat16), (((1,), (1,)), ((), ())),
        preferred_element_type=jnp.float32,
    ) + gb_ref[...]
    m = jnp.max(logits, axis=1, keepdims=True)
    scale = 1.0 / jnp.sum(jnp.exp(logits - m), axis=1, keepdims=True)
    out = jax.lax.dot_general(
        xb, w_ref[...].astype(jnp.bfloat16), (((1,), (1,)), ((), ())),
        preferred_element_type=jnp.float32,
    )
    o_ref[...] = (out + b_ref[...]) * scale


def _run(hs, gate_w, gate_b2, expert_w, expert_b2, *, tile_n, interpret=False):
    n, d = hs.shape
    e = gate_w.shape[0]
    grid = (n // tile_n,)
    return pl.pallas_call(
        _fused_moe_kernel,
        grid=grid,
        in_specs=[
            pl.BlockSpec((tile_n, d), lambda i: (i, 0)),
            pl.BlockSpec((e, d), lambda i: (0, 0)),
            pl.BlockSpec((1, e), lambda i: (0, 0)),
            pl.BlockSpec((d, d), lambda i: (0, 0)),
            pl.BlockSpec((1, d), lambda i: (0, 0)),
        ],
        out_specs=pl.BlockSpec((tile_n, d), lambda i: (i, 0)),
        out_shape=jax.ShapeDtypeStruct((n, d), jnp.float32),
        compiler_params=pltpu.CompilerParams(
            dimension_semantics=("parallel",),
        ),
        interpret=interpret,
    )(hs, gate_w, gate_b2, expert_w, expert_b2)


def kernel(x, gate_w, gate_b, expert_w, expert_b):
    b, s, d = x.shape
    hs = x.reshape(b * s, d)
    out = _run(
        hs,
        gate_w,
        gate_b.reshape(1, -1),
        expert_w,
        expert_b.reshape(1, -1),
        tile_n=512,
    )
    return out.reshape(b, s, d)
